# Initial kernel scaffold; baseline (speedup 1.0000x reference)
import jax
import jax.numpy as jnp
from jax.experimental import pallas as pl


def kernel(hid, choices, tags, e_weight):
    raise NotImplementedError("write your pallas kernel here")


# speedup vs baseline: 1.2301x; 1.2301x over previous
import functools

import jax
import jax.numpy as jnp
from jax import lax
from jax.experimental import pallas as pl
from jax.experimental.pallas import tpu as pltpu
from jax.experimental.pallas import tpu_sc as plsc

_NC = 2
_NS = 16
_NW = _NC * _NS
_L = 16


@functools.partial(jax.jit, static_argnums=())
def _sc_pred(e_weight, hid2, ch2):
    N, D = hid2.shape
    C = ch2.shape[1]
    tpw = N // _NW
    mesh = plsc.VectorSubcoreMesh(core_axis_name="c", subcore_axis_name="s")

    @functools.partial(
        pl.kernel,
        out_type=jax.ShapeDtypeStruct((N, C), jnp.float32),
        mesh=mesh,
        scratch_types=[
            pltpu.VMEM((tpw, C), jnp.int32),
            pltpu.VMEM((tpw, D), jnp.float32),
            pltpu.VMEM((C, D), jnp.float32),
            pltpu.VMEM((tpw, C), jnp.float32),
            pltpu.SemaphoreType.DMA,
        ],
        compiler_params=pltpu.CompilerParams(
            use_tc_tiling_on_sc=False, needs_layout_passes=False),
    )
    def k(table_hbm, hid_hbm, ch_hbm, out_hbm, ch_v, hid_v, rows_v, pred_v, sem):
        wid = lax.axis_index("s") * _NC + lax.axis_index("c")
        base = wid * tpw
        pltpu.sync_copy(ch_hbm.at[pl.ds(base, tpw)], ch_v)
        pltpu.sync_copy(hid_hbm.at[pl.ds(base, tpw)], hid_v)

        lane = lax.iota(jnp.int32, _L)
        _A = 4
        _NG = C // _L
        zeros = jnp.zeros((_L,), jnp.float32)
        row_idx = [lane + g * _L for g in range(_NG)]

        def tok_body(t, carry):
            pltpu.async_copy(table_hbm.at[ch_v.at[t]], rows_v, sem).wait()

            def d_body(dc, accs):
                accs = list(accs)
                h = hid_v[t, pl.ds(dc * _L, _L)]
                hu = plsc.bitcast(h, jnp.uint32)
                hu = ((hu + jnp.uint32(0x7FFF) + ((hu >> 16) & jnp.uint32(1)))
                      & jnp.uint32(0xFFFF0000))
                h = plsc.bitcast(hu, jnp.float32)
                colv = jnp.full((_L,), dc * _L, jnp.int32)
                for u in range(_L):
                    col = colv + u
                    for g in range(_NG):
                        w = plsc.load_gather(rows_v, [row_idx[g], col])
                        wu = plsc.bitcast(w, jnp.uint32)
                        wu = ((wu + jnp.uint32(0x7FFF) + ((wu >> 16) & jnp.uint32(1)))
                              & jnp.uint32(0xFFFF0000))
                        w = plsc.bitcast(wu, jnp.float32)
                        a = g * _A + (u % _A)
                        accs[a] = accs[a] + w * h[u]
                return tuple(accs)

            accs = lax.fori_loop(0, D // _L, d_body, (zeros,) * (_A * _NG),
                                 unroll=False)
            for g in range(_NG):
                a = accs[g * _A: (g + 1) * _A]
                pred_v[t, pl.ds(g * _L, _L)] = (a[0] + a[1]) + (a[2] + a[3])
            return carry

        lax.fori_loop(0, tpw, tok_body, 0, unroll=False)
        pltpu.sync_copy(pred_v, out_hbm.at[pl.ds(base, tpw)])

    return k(e_weight, hid2, ch2)


def _tail(pred, ch2, tags2):
    N, C = pred.shape

    def body(p_ref, c_ref, t_ref, loss_ref, y_ref):
        p = p_ref[...]
        ch = c_ref[...]
        tg = t_ref[...]
        iota = lax.broadcasted_iota(jnp.int32, (N, C), 1)
        m = jnp.max(p, axis=1, keepdims=True)
        am = jnp.min(jnp.where(p == m, iota, C), axis=1, keepdims=True)
        y_ref[...] = jnp.sum(jnp.where(iota == am, ch, 0), axis=1, keepdims=True)
        tval = jnp.sum(jnp.where(iota == tg, p, 0.0), axis=1, keepdims=True)
        s = jnp.sum(jnp.exp(p - m), axis=1, keepdims=True)
        nll = m + jnp.log(s) - tval
        loss_ref[...] = jnp.broadcast_to(jnp.sum(nll) / N, (1, 1))

    return pl.pallas_call(
        body,
        out_shape=(
            jax.ShapeDtypeStruct((1, 1), jnp.float32),
            jax.ShapeDtypeStruct((N, 1), jnp.int32),
        ),
    )(pred, ch2, tags2)


def kernel(hid, choices, tags, e_weight):
    B, T, D = hid.shape
    C = choices.shape[-1]
    N = B * T
    hid2 = hid.reshape(N, D)
    ch2 = choices.reshape(N, C)
    pred = _sc_pred(e_weight, hid2, ch2)
    loss, y = _tail(pred, ch2, tags.reshape(N, 1))
    return loss[0, 0], y.reshape(B, T)


# speedup vs baseline: 1.3152x; 1.0691x over previous
import functools

import jax
import jax.numpy as jnp
from jax import lax
from jax.experimental import pallas as pl
from jax.experimental.pallas import tpu as pltpu
from jax.experimental.pallas import tpu_sc as plsc

_NC = 2
_NS = 16
_NW = _NC * _NS
_L = 16
_NB = 4


@functools.partial(jax.jit, static_argnums=())
def _sc_pred(e_weight, hid2, ch2):
    N, D = hid2.shape
    C = ch2.shape[1]
    tpw = N // _NW
    mesh = plsc.VectorSubcoreMesh(core_axis_name="c", subcore_axis_name="s")

    @functools.partial(
        pl.kernel,
        out_type=jax.ShapeDtypeStruct((N, C), jnp.float32),
        mesh=mesh,
        scratch_types=[
            pltpu.VMEM((tpw, C), jnp.int32),
            pltpu.VMEM((tpw, D), jnp.float32),
            pltpu.VMEM((_NB, C, D), jnp.float32),
            pltpu.VMEM((tpw, C), jnp.float32),
        ] + [pltpu.SemaphoreType.DMA] * _NB,
        compiler_params=pltpu.CompilerParams(
            use_tc_tiling_on_sc=False, needs_layout_passes=False),
    )
    def k(table_hbm, hid_hbm, ch_hbm, out_hbm, ch_v, hid_v, rows_v, pred_v,
          *sems):
        wid = lax.axis_index("s") * _NC + lax.axis_index("c")
        base = wid * tpw
        pltpu.sync_copy(ch_hbm.at[pl.ds(base, tpw)], ch_v)
        pltpu.sync_copy(hid_hbm.at[pl.ds(base, tpw)], hid_v)

        lane = lax.iota(jnp.int32, _L)
        _A = 4
        _NG = C // _L
        zeros = jnp.zeros((_L,), jnp.float32)
        row_idx = [lane + g * _L for g in range(_NG)]

        for b in range(_NB):
            pltpu.async_copy(table_hbm.at[ch_v.at[b]], rows_v.at[b], sems[b])

        def compute_tok(t, b):
            def d_body(dc, accs):
                accs = list(accs)
                h = hid_v[t, pl.ds(dc * _L, _L)]
                hu = plsc.bitcast(h, jnp.uint32)
                hu = ((hu + jnp.uint32(0x7FFF) + ((hu >> 16) & jnp.uint32(1)))
                      & jnp.uint32(0xFFFF0000))
                h = plsc.bitcast(hu, jnp.float32)
                colv = jnp.full((_L,), dc * _L, jnp.int32)
                for u in range(_L):
                    col = colv + u
                    for g in range(_NG):
                        w = plsc.load_gather(rows_v.at[b], [row_idx[g], col])
                        wu = plsc.bitcast(w, jnp.uint32)
                        wu = ((wu + jnp.uint32(0x7FFF)
                               + ((wu >> 16) & jnp.uint32(1)))
                              & jnp.uint32(0xFFFF0000))
                        w = plsc.bitcast(wu, jnp.float32)
                        a = g * _A + (u % _A)
                        accs[a] = accs[a] + w * h[u]
                return tuple(accs)

            accs = lax.fori_loop(0, D // _L, d_body, (zeros,) * (_A * _NG),
                                 unroll=False)
            for g in range(_NG):
                a = accs[g * _A: (g + 1) * _A]
                pred_v[t, pl.ds(g * _L, _L)] = (a[0] + a[1]) + (a[2] + a[3])

        def ring_body(p, carry):
            for b in range(_NB):
                t = p * _NB + b
                pltpu.make_async_copy(
                    table_hbm.at[pl.ds(0, C)], rows_v.at[b], sems[b]).wait()
                compute_tok(t, b)

                @pl.when(t + _NB < tpw)
                def _():
                    pltpu.async_copy(
                        table_hbm.at[ch_v.at[t + _NB]], rows_v.at[b], sems[b])
            return carry

        lax.fori_loop(0, tpw // _NB, ring_body, 0, unroll=False)
        pltpu.sync_copy(pred_v, out_hbm.at[pl.ds(base, tpw)])

    return k(e_weight, hid2, ch2)


def _tail(pred, ch2, tags2):
    N, C = pred.shape

    def body(p_ref, c_ref, t_ref, loss_ref, y_ref):
        p = p_ref[...]
        ch = c_ref[...]
        tg = t_ref[...]
        iota = lax.broadcasted_iota(jnp.int32, (N, C), 1)
        m = jnp.max(p, axis=1, keepdims=True)
        am = jnp.min(jnp.where(p == m, iota, C), axis=1, keepdims=True)
        y_ref[...] = jnp.sum(jnp.where(iota == am, ch, 0), axis=1, keepdims=True)
        tval = jnp.sum(jnp.where(iota == tg, p, 0.0), axis=1, keepdims=True)
        s = jnp.sum(jnp.exp(p - m), axis=1, keepdims=True)
        nll = m + jnp.log(s) - tval
        loss_ref[...] = jnp.broadcast_to(jnp.sum(nll) / N, (1, 1))

    return pl.pallas_call(
        body,
        out_shape=(
            jax.ShapeDtypeStruct((1, 1), jnp.float32),
            jax.ShapeDtypeStruct((N, 1), jnp.int32),
        ),
    )(pred, ch2, tags2)


def kernel(hid, choices, tags, e_weight):
    B, T, D = hid.shape
    C = choices.shape[-1]
    N = B * T
    hid2 = hid.reshape(N, D)
    ch2 = choices.reshape(N, C)
    pred = _sc_pred(e_weight, hid2, ch2)
    loss, y = _tail(pred, ch2, tags.reshape(N, 1))
    return loss[0, 0], y.reshape(B, T)


# speedup vs baseline: 2.2990x; 1.7480x over previous
import functools

import jax
import jax.numpy as jnp
from jax import lax
from jax.experimental import pallas as pl
from jax.experimental.pallas import tpu as pltpu
from jax.experimental.pallas import tpu_sc as plsc

_NC = 2
_NS = 16
_NW = _NC * _NS
_L = 16
_NB = 4


@functools.partial(jax.jit, static_argnums=())
def _sc_pred(e_weight, hid2, ch2):
    N, D = hid2.shape
    C = ch2.shape[1]
    tpw = N // _NW
    mesh = plsc.VectorSubcoreMesh(core_axis_name="c", subcore_axis_name="s")

    @functools.partial(
        pl.kernel,
        out_type=jax.ShapeDtypeStruct((N, C), jnp.float32),
        mesh=mesh,
        scratch_types=[
            pltpu.VMEM((tpw, C), jnp.int32),
            pltpu.VMEM((tpw, D), jnp.float32),
            pltpu.VMEM((_NB, C, D), jnp.float32),
            pltpu.VMEM((tpw, C), jnp.float32),
        ] + [pltpu.SemaphoreType.DMA] * _NB,
        compiler_params=pltpu.CompilerParams(
            use_tc_tiling_on_sc=False, needs_layout_passes=False),
    )
    def k(table_hbm, hid_hbm, ch_hbm, out_hbm, ch_v, hid_v, rows_v, pred_v,
          *sems):
        wid = lax.axis_index("s") * _NC + lax.axis_index("c")
        base = wid * tpw
        pltpu.sync_copy(ch_hbm.at[pl.ds(base, tpw)], ch_v)
        pltpu.sync_copy(hid_hbm.at[pl.ds(base, tpw)], hid_v)

        lane = lax.iota(jnp.int32, _L)
        _A = 4
        _NK = D // _L
        zeros = jnp.zeros((_L,), jnp.float32)

        def _rne(v):
            u = plsc.bitcast(v, jnp.uint32)
            u = ((u + jnp.uint32(0x7FFF) + ((u >> 16) & jnp.uint32(1)))
                 & jnp.uint32(0xFFFF0000))
            return plsc.bitcast(u, jnp.float32)

        def hid_rne_body(t, carry):
            for kk in range(_NK):
                sl = pl.ds(kk * _L, _L)
                hid_v[t, sl] = _rne(hid_v[t, sl])
            return carry

        lax.fori_loop(0, tpw, hid_rne_body, 0, unroll=False)

        for b in range(_NB):
            pltpu.async_copy(table_hbm.at[ch_v.at[b]], rows_v.at[b], sems[b])

        def compute_tok(t, b):
            def cand_body(c, carry):
                vec0, vec1 = carry
                accs = [zeros] * _A
                for kk in range(_NK):
                    w = _rne(rows_v[b, c, pl.ds(kk * _L, _L)])
                    h = hid_v[t, pl.ds(kk * _L, _L)]
                    accs[kk % _A] = accs[kk % _A] + w * h
                tot = jnp.sum((accs[0] + accs[1]) + (accs[2] + accs[3]))
                hit = lane == (c & (_L - 1))
                vec0 = jnp.where(hit & (c < _L), tot, vec0)
                vec1 = jnp.where(hit & (c >= _L), tot, vec1)
                return vec0, vec1

            vec0, vec1 = lax.fori_loop(0, C, cand_body, (zeros, zeros),
                                       unroll=False)
            pred_v[t, pl.ds(0, _L)] = vec0
            pred_v[t, pl.ds(_L, _L)] = vec1

        def ring_body(p, carry):
            for b in range(_NB):
                t = p * _NB + b
                pltpu.make_async_copy(
                    table_hbm.at[pl.ds(0, C)], rows_v.at[b], sems[b]).wait()
                compute_tok(t, b)

                @pl.when(t + _NB < tpw)
                def _():
                    pltpu.async_copy(
                        table_hbm.at[ch_v.at[t + _NB]], rows_v.at[b], sems[b])
            return carry

        lax.fori_loop(0, tpw // _NB, ring_body, 0, unroll=False)
        pltpu.sync_copy(pred_v, out_hbm.at[pl.ds(base, tpw)])

    return k(e_weight, hid2, ch2)


def _tail(pred, ch2, tags2):
    N, C = pred.shape

    def body(p_ref, c_ref, t_ref, loss_ref, y_ref):
        p = p_ref[...]
        ch = c_ref[...]
        tg = t_ref[...]
        iota = lax.broadcasted_iota(jnp.int32, (N, C), 1)
        m = jnp.max(p, axis=1, keepdims=True)
        am = jnp.min(jnp.where(p == m, iota, C), axis=1, keepdims=True)
        y_ref[...] = jnp.sum(jnp.where(iota == am, ch, 0), axis=1, keepdims=True)
        tval = jnp.sum(jnp.where(iota == tg, p, 0.0), axis=1, keepdims=True)
        s = jnp.sum(jnp.exp(p - m), axis=1, keepdims=True)
        nll = m + jnp.log(s) - tval
        loss_ref[...] = jnp.broadcast_to(jnp.sum(nll) / N, (1, 1))

    return pl.pallas_call(
        body,
        out_shape=(
            jax.ShapeDtypeStruct((1, 1), jnp.float32),
            jax.ShapeDtypeStruct((N, 1), jnp.int32),
        ),
    )(pred, ch2, tags2)


def kernel(hid, choices, tags, e_weight):
    B, T, D = hid.shape
    C = choices.shape[-1]
    N = B * T
    hid2 = hid.reshape(N, D)
    ch2 = choices.reshape(N, C)
    pred = _sc_pred(e_weight, hid2, ch2)
    loss, y = _tail(pred, ch2, tags.reshape(N, 1))
    return loss[0, 0], y.reshape(B, T)
